# baseline (speedup 1.0000x reference)
import functools
import math

import jax
import jax.numpy as jnp
from jax.experimental import pallas as pl
from jax.experimental.pallas import tpu as pltpu

_D = 2048
_E = 64
_TOP_K = 8
_CHUNK = 128
_STEPS = _D // _CHUNK
_SCALE = 1.0 / math.sqrt(_D)
_HI = jax.lax.Precision.HIGHEST


def _router_body(c_ref, keys_ref, wq_ref, wk_ref, out_ref, t_ref):
    i = pl.program_id(0)

    @pl.when(i == 0)
    def _init():
        t_ref[...] = jnp.zeros_like(t_ref)

    q_i = jnp.sum(wq_ref[...] * c_ref[...], axis=1, keepdims=True)
    t_ref[...] += jnp.sum(q_i * wk_ref[...], axis=0, keepdims=True)

    @pl.when(i == _STEPS - 1)
    def _finish():
        t = t_ref[...]
        scores = jax.lax.dot_general(
            t, keys_ref[...], (((1,), (1,)), ((), ())),
            precision=_HI, preferred_element_type=jnp.float32) * _SCALE
        lane = jax.lax.broadcasted_iota(jnp.int32, (1, _E), 1)
        neg_inf = jnp.float32(-jnp.inf)
        selected = jnp.zeros((1, _E), dtype=jnp.bool_)
        avail = scores
        for _ in range(_TOP_K):
            m = jnp.max(avail)
            idx = jnp.min(jnp.where(avail == m, lane, _E))
            selected = selected | (lane == idx)
            avail = jnp.where(selected, neg_inf, scores)
        logits = jnp.where(selected, scores, neg_inf)
        mx = jnp.max(logits)
        ex = jnp.where(selected, jnp.exp(logits - mx), 0.0)
        out_ref[...] = ex / jnp.sum(ex)


@functools.partial(jax.jit, static_argnames=("interpret",))
def _router(context, keys, Wq_weight, Wk_weight, interpret=False):
    c2 = context.reshape(1, _D)
    gate = pl.pallas_call(
        _router_body,
        grid=(_STEPS,),
        in_specs=[
            pl.BlockSpec((1, _D), lambda i: (0, 0)),
            pl.BlockSpec((_E, _D), lambda i: (0, 0)),
            pl.BlockSpec((_CHUNK, _D), lambda i: (i, 0)),
            pl.BlockSpec((_CHUNK, _D), lambda i: (i, 0)),
        ],
        out_specs=pl.BlockSpec((1, _E), lambda i: (0, 0)),
        out_shape=jax.ShapeDtypeStruct((1, _E), jnp.float32),
        scratch_shapes=[pltpu.VMEM((1, _D), jnp.float32)],
        interpret=interpret,
    )(c2, keys, Wq_weight, Wk_weight)
    return gate.reshape(_E)


def kernel(expert_outputs, context, keys, Wq_weight, Wk_weight):
    del expert_outputs
    return _router(context, keys, Wq_weight, Wk_weight)


# speedup vs baseline: 1.2559x; 1.2559x over previous
import functools
import math

import jax
import jax.numpy as jnp
from jax.experimental import pallas as pl
from jax.experimental.pallas import tpu as pltpu

_D = 2048
_E = 64
_TOP_K = 8
_CHUNK = 1024
_STEPS = _D // _CHUNK
_SCALE = 1.0 / math.sqrt(_D)
_HI = jax.lax.Precision.HIGHEST


def _router_body(c_ref, keys_ref, wq_ref, wk_ref, out_ref, t_ref):
    i = pl.program_id(0)

    @pl.when(i == 0)
    def _init():
        t_ref[...] = jnp.zeros_like(t_ref)

    q_i = jnp.sum(wq_ref[...] * c_ref[...], axis=1, keepdims=True)
    t_ref[...] += jnp.sum(q_i * wk_ref[...], axis=0, keepdims=True)

    @pl.when(i == _STEPS - 1)
    def _finish():
        t = t_ref[...]
        scores = jax.lax.dot_general(
            t, keys_ref[...], (((1,), (1,)), ((), ())),
            precision=_HI, preferred_element_type=jnp.float32) * _SCALE
        lane = jax.lax.broadcasted_iota(jnp.int32, (1, _E), 1)
        neg_inf = jnp.float32(-jnp.inf)
        selected = jnp.zeros((1, _E), dtype=jnp.bool_)
        avail = scores
        for _ in range(_TOP_K):
            m = jnp.max(avail)
            idx = jnp.min(jnp.where(avail == m, lane, _E))
            selected = selected | (lane == idx)
            avail = jnp.where(selected, neg_inf, scores)
        logits = jnp.where(selected, scores, neg_inf)
        mx = jnp.max(logits)
        ex = jnp.where(selected, jnp.exp(logits - mx), 0.0)
        out_ref[...] = ex / jnp.sum(ex)


@functools.partial(jax.jit, static_argnames=("interpret",))
def _router(context, keys, Wq_weight, Wk_weight, interpret=False):
    c2 = context.reshape(1, _D)
    gate = pl.pallas_call(
        _router_body,
        grid=(_STEPS,),
        in_specs=[
            pl.BlockSpec((1, _D), lambda i: (0, 0)),
            pl.BlockSpec((_E, _D), lambda i: (0, 0)),
            pl.BlockSpec((_CHUNK, _D), lambda i: (i, 0)),
            pl.BlockSpec((_CHUNK, _D), lambda i: (i, 0)),
        ],
        out_specs=pl.BlockSpec((1, _E), lambda i: (0, 0)),
        out_shape=jax.ShapeDtypeStruct((1, _E), jnp.float32),
        scratch_shapes=[pltpu.VMEM((1, _D), jnp.float32)],
        interpret=interpret,
    )(c2, keys, Wq_weight, Wk_weight)
    return gate.reshape(_E)


def kernel(expert_outputs, context, keys, Wq_weight, Wk_weight):
    del expert_outputs
    return _router(context, keys, Wq_weight, Wk_weight)
